# baseline (speedup 1.0000x reference)
import functools

import jax
import jax.numpy as jnp
from jax import lax
from jax.experimental import pallas as pl
from jax.experimental.pallas import tpu as pltpu
from jax.experimental.pallas import tpu_sc as plsc

B = 262144
C = 32
D = H = W = 128
DHW = D * H * W

NC = 2
NS = 16
NW = NC * NS
PW = B // NW
P = 128
NCHUNK = PW // P
L = 16


def _axis_coords(p):
    f = ((p + 1.0) * 0.5) * 127.0
    i0 = f.astype(jnp.int32)
    fr = f - i0.astype(jnp.float32)
    i1 = jnp.minimum(i0 + 1, 127)
    return i0, i1, fr


mesh = plsc.VectorSubcoreMesh(core_axis_name="c", subcore_axis_name="s")

TV = 1024
TCHUNK = DHW // NW // TV


@functools.partial(
    pl.kernel,
    out_type=jax.ShapeDtypeStruct((DHW, C), jnp.float32),
    mesh=mesh,
    scratch_types=[
        pltpu.VMEM((C * TV,), jnp.float32),
        pltpu.VMEM((C * TV,), jnp.float32),
        pltpu.VMEM((TV, C), jnp.float32),
        pltpu.SemaphoreType.DMA,
        pltpu.SemaphoreType.DMA,
    ],
    compiler_params=pltpu.CompilerParams(use_tc_tiling_on_sc=False,
                                         needs_layout_passes=False),
)
def _sc_build(vox_hbm, table_hbm, cbuf0, cbuf1, rbuf, sem0, sem1):
    wid = lax.axis_index("s") * NC + lax.axis_index("c")
    v0 = wid * (DHW // NW)
    lanes = lax.iota(jnp.int32, L)

    def fire(g, buf, sem):
        base = v0 + g * TV
        for c in range(C):
            pltpu.async_copy(vox_hbm.at[pl.ds(c * DHW + base, TV)],
                             buf.at[pl.ds(c * TV, TV)], sem)

    def drain(g, buf, sem):
        base = v0 + g * TV
        for c in range(C):
            pltpu.make_async_copy(vox_hbm.at[pl.ds(c * DHW + base, TV)],
                                  buf.at[pl.ds(c * TV, TV)], sem).wait()

    fire(0, cbuf0, sem0)
    bufs = (cbuf0, cbuf1)
    sems = (sem0, sem1)

    def chunk2(i, carry):
        for b in range(2):
            gg = i * 2 + b
            buf, sem = bufs[b], sems[b]

            @pl.when(gg + 1 < TCHUNK)
            def _():
                fire(gg + 1, bufs[1 - b], sems[1 - b])

            drain(gg, buf, sem)

            coff = lanes * TV

            def vstep(t, carry2):
                vb = t * L
                for j in range(L):
                    v = vb + j
                    lo = plsc.load_gather(buf, [coff + v])
                    hi = plsc.load_gather(buf, [coff + (16 * TV + v)])
                    rbuf[v, pl.ds(0, L)] = lo
                    rbuf[v, pl.ds(L, L)] = hi
                return carry2

            lax.fori_loop(0, TV // L, vstep, 0)
            pltpu.sync_copy(rbuf, table_hbm.at[pl.ds(v0 + gg * TV, TV)])
        return carry

    lax.fori_loop(0, TCHUNK // 2, chunk2, 0)


@functools.partial(
    pl.kernel,
    out_type=jax.ShapeDtypeStruct((B, C), jnp.float32),
    mesh=mesh,
    scratch_types=[
        pltpu.VMEM((P,), jnp.float32),
        pltpu.VMEM((P,), jnp.float32),
        pltpu.VMEM((P,), jnp.float32),
        pltpu.VMEM((8, P), jnp.int32),
        pltpu.VMEM((8 * P,), jnp.float32),
        pltpu.VMEM((8 * P, C), jnp.float32),
        pltpu.VMEM((P, C), jnp.float32),
        pltpu.SemaphoreType.DMA,
    ],
    compiler_params=pltpu.CompilerParams(use_tc_tiling_on_sc=False),
)
def _sc_interp(pts_hbm, table_hbm, out_hbm,
               zv, yv, xv, idx_v, w8_v, rows_v, out_v, gsem):
    wid = lax.axis_index("s") * NC + lax.axis_index("c")
    base = wid * PW

    def chunk_body(g, carry):
        row0 = base + g * P
        pltpu.sync_copy(pts_hbm.at[pl.ds(row0, P)], zv)
        pltpu.sync_copy(pts_hbm.at[pl.ds(B + row0, P)], yv)
        pltpu.sync_copy(pts_hbm.at[pl.ds(2 * B + row0, P)], xv)

        for t in range(P // L):
            s = t * L
            sl = pl.ds(s, L)
            zi0, zi1, fz = _axis_coords(zv[sl])
            yi0, yi1, fy = _axis_coords(yv[sl])
            xi0, xi1, fx = _axis_coords(xv[sl])
            zy00 = zi0 * (H * W) + yi0 * W
            zy01 = zi0 * (H * W) + yi1 * W
            zy10 = zi1 * (H * W) + yi0 * W
            zy11 = zi1 * (H * W) + yi1 * W
            idx_v[0, sl] = zy00 + xi0
            idx_v[1, sl] = zy00 + xi1
            idx_v[2, sl] = zy01 + xi0
            idx_v[3, sl] = zy01 + xi1
            idx_v[4, sl] = zy10 + xi0
            idx_v[5, sl] = zy10 + xi1
            idx_v[6, sl] = zy11 + xi0
            idx_v[7, sl] = zy11 + xi1
            fz0 = 1.0 - fz
            fy0 = 1.0 - fy
            fx0 = 1.0 - fx
            m00 = fz0 * fy0
            m01 = fz0 * fy
            m10 = fz * fy0
            m11 = fz * fy
            w8_v[pl.ds(0 * P + s, L)] = m00 * fx0
            w8_v[pl.ds(1 * P + s, L)] = m00 * fx
            w8_v[pl.ds(2 * P + s, L)] = m01 * fx0
            w8_v[pl.ds(3 * P + s, L)] = m01 * fx
            w8_v[pl.ds(4 * P + s, L)] = m10 * fx0
            w8_v[pl.ds(5 * P + s, L)] = m10 * fx
            w8_v[pl.ds(6 * P + s, L)] = m11 * fx0
            w8_v[pl.ds(7 * P + s, L)] = m11 * fx

        copies = [
            pltpu.async_copy(table_hbm.at[idx_v.at[k]],
                             rows_v.at[pl.ds(k * P, P)], gsem)
            for k in range(8)
        ]
        for cp in copies:
            cp.wait()

        def grp_body(t, carry2):
            jbase = t * L
            wrows = [w8_v[pl.ds(k * P + jbase, L)] for k in range(8)]
            for jj in range(L):
                j = jbase + jj
                lane = jnp.full((L,), jj, jnp.int32)
                acc0 = jnp.zeros((L,), jnp.float32)
                acc1 = jnp.zeros((L,), jnp.float32)
                for k in range(8):
                    wb = wrows[k][lane]
                    acc0 = acc0 + wb * rows_v[k * P + j, pl.ds(0, L)]
                    acc1 = acc1 + wb * rows_v[k * P + j, pl.ds(L, L)]
                out_v[j, pl.ds(0, L)] = acc0
                out_v[j, pl.ds(L, L)] = acc1
            return carry2

        lax.fori_loop(0, P // L, grp_body, 0)
        pltpu.sync_copy(out_v, out_hbm.at[pl.ds(row0, P)])
        return carry

    lax.fori_loop(0, NCHUNK, chunk_body, 0)


def kernel(warped_sample_points, voxel_grid):
    vox_flat = voxel_grid.reshape(C * DHW)
    table = _sc_build(vox_flat)
    pts = warped_sample_points.T.reshape(3 * B)
    return _sc_interp(pts, table)


# speedup vs baseline: 1.3859x; 1.3859x over previous
import functools

import jax
import jax.numpy as jnp
from jax import lax
from jax.experimental import pallas as pl
from jax.experimental.pallas import tpu as pltpu
from jax.experimental.pallas import tpu_sc as plsc

B = 262144
C = 32
D = H = W = 128
DHW = D * H * W

NC = 2
NS = 16
NW = NC * NS
PW = B // NW
P = 128
NCHUNK = PW // P
L = 16


def _axis_coords(p):
    f = ((p + 1.0) * 0.5) * 127.0
    i0 = f.astype(jnp.int32)
    fr = f - i0.astype(jnp.float32)
    i1 = jnp.minimum(i0 + 1, 127)
    return i0, i1, fr


mesh = plsc.VectorSubcoreMesh(core_axis_name="c", subcore_axis_name="s")

@functools.partial(
    pl.kernel,
    out_type=jax.ShapeDtypeStruct((B, C), jnp.float32),
    mesh=mesh,
    scratch_types=[
        pltpu.VMEM((3 * P,), jnp.float32),
        pltpu.VMEM((8, P), jnp.int32),
        pltpu.VMEM((8 * P,), jnp.float32),
        pltpu.VMEM((8 * P, C), jnp.float32),
        pltpu.VMEM((P, C), jnp.float32),
        pltpu.SemaphoreType.DMA,
    ],
    compiler_params=pltpu.CompilerParams(use_tc_tiling_on_sc=False),
)
def _sc_interp(pts_hbm, table_hbm, out_hbm,
               pbuf, idx_v, w8_v, rows_v, out_v, gsem):
    wid = lax.axis_index("s") * NC + lax.axis_index("c")
    base = wid * PW

    lanes3 = lax.iota(jnp.int32, L) * 3
    idxm = [(lanes3 + c0) & (L - 1) for c0 in range(3)]
    vsel = [(lanes3 + c0) >> 4 for c0 in range(3)]
    sel0 = [v == 0 for v in vsel]
    sel1 = [v == 1 for v in vsel]

    def chunk_body(g, carry):
        row0 = base + g * P
        pltpu.sync_copy(pts_hbm.at[pl.ds(row0 * 3, P * 3)], pbuf)

        for t in range(P // L):
            s = t * L
            sl = pl.ds(s, L)
            v0 = pbuf[pl.ds(t * 48, L)]
            v1 = pbuf[pl.ds(t * 48 + L, L)]
            v2 = pbuf[pl.ds(t * 48 + 2 * L, L)]

            def deint(c0):
                g0 = v0[idxm[c0]]
                g1 = v1[idxm[c0]]
                g2 = v2[idxm[c0]]
                return jnp.where(sel0[c0], g0, jnp.where(sel1[c0], g1, g2))

            zi0, zi1, fz = _axis_coords(deint(0))
            yi0, yi1, fy = _axis_coords(deint(1))
            xi0, xi1, fx = _axis_coords(deint(2))
            zy00 = zi0 * (H * W) + yi0 * W
            zy01 = zi0 * (H * W) + yi1 * W
            zy10 = zi1 * (H * W) + yi0 * W
            zy11 = zi1 * (H * W) + yi1 * W
            idx_v[0, sl] = zy00 + xi0
            idx_v[1, sl] = zy00 + xi1
            idx_v[2, sl] = zy01 + xi0
            idx_v[3, sl] = zy01 + xi1
            idx_v[4, sl] = zy10 + xi0
            idx_v[5, sl] = zy10 + xi1
            idx_v[6, sl] = zy11 + xi0
            idx_v[7, sl] = zy11 + xi1
            fz0 = 1.0 - fz
            fy0 = 1.0 - fy
            fx0 = 1.0 - fx
            m00 = fz0 * fy0
            m01 = fz0 * fy
            m10 = fz * fy0
            m11 = fz * fy
            w8_v[pl.ds(0 * P + s, L)] = m00 * fx0
            w8_v[pl.ds(1 * P + s, L)] = m00 * fx
            w8_v[pl.ds(2 * P + s, L)] = m01 * fx0
            w8_v[pl.ds(3 * P + s, L)] = m01 * fx
            w8_v[pl.ds(4 * P + s, L)] = m10 * fx0
            w8_v[pl.ds(5 * P + s, L)] = m10 * fx
            w8_v[pl.ds(6 * P + s, L)] = m11 * fx0
            w8_v[pl.ds(7 * P + s, L)] = m11 * fx

        copies = [
            pltpu.async_copy(table_hbm.at[idx_v.at[k]],
                             rows_v.at[pl.ds(k * P, P)], gsem)
            for k in range(8)
        ]
        for cp in copies:
            cp.wait()

        def grp_body(t, carry2):
            jbase = t * L
            wrows = [w8_v[pl.ds(k * P + jbase, L)] for k in range(8)]
            for jj in range(L):
                j = jbase + jj
                lane = jnp.full((L,), jj, jnp.int32)
                acc0 = jnp.zeros((L,), jnp.float32)
                acc1 = jnp.zeros((L,), jnp.float32)
                for k in range(8):
                    wb = wrows[k][lane]
                    acc0 = acc0 + wb * rows_v[k * P + j, pl.ds(0, L)]
                    acc1 = acc1 + wb * rows_v[k * P + j, pl.ds(L, L)]
                out_v[j, pl.ds(0, L)] = acc0
                out_v[j, pl.ds(L, L)] = acc1
            return carry2

        lax.fori_loop(0, P // L, grp_body, 0)
        pltpu.sync_copy(out_v, out_hbm.at[pl.ds(row0, P)])
        return carry

    lax.fori_loop(0, NCHUNK, chunk_body, 0)


def kernel(warped_sample_points, voxel_grid):
    table = voxel_grid[0].transpose(1, 2, 3, 0).reshape(DHW, C)
    pts = warped_sample_points.reshape(3 * B)
    return _sc_interp(pts, table)


# speedup vs baseline: 1.3872x; 1.0009x over previous
import functools

import jax
import jax.numpy as jnp
from jax import lax
from jax.experimental import pallas as pl
from jax.experimental.pallas import tpu as pltpu
from jax.experimental.pallas import tpu_sc as plsc

B = 262144
C = 32
D = H = W = 128
DHW = D * H * W

NC = 2
NS = 16
NW = NC * NS
PW = B // NW
P = 128
NCHUNK = PW // P
L = 16


def _axis_coords(p):
    f = ((p + 1.0) * 0.5) * 127.0
    i0 = f.astype(jnp.int32)
    fr = f - i0.astype(jnp.float32)
    i1 = jnp.minimum(i0 + 1, 127)
    return i0, i1, fr


mesh = plsc.VectorSubcoreMesh(core_axis_name="c", subcore_axis_name="s")

@functools.partial(
    pl.kernel,
    out_type=jax.ShapeDtypeStruct((B, C), jnp.float32),
    mesh=mesh,
    scratch_types=[
        pltpu.VMEM((3 * P,), jnp.float32),
        pltpu.VMEM((8, P), jnp.int32),
        pltpu.VMEM((8 * P,), jnp.float32),
        pltpu.VMEM((8 * P, C), jnp.float32),
        pltpu.VMEM((P, C), jnp.float32),
        pltpu.SemaphoreType.DMA,
    ],
    compiler_params=pltpu.CompilerParams(use_tc_tiling_on_sc=False),
)
def _sc_interp(pts_hbm, table_hbm, out_hbm,
               pbuf, idx_v, w8_v, rows_v, out_v, gsem):
    wid = lax.axis_index("s") * NC + lax.axis_index("c")
    base = wid * PW

    lanes3 = lax.iota(jnp.int32, L) * 3
    idxm = [(lanes3 + c0) & (L - 1) for c0 in range(3)]
    vsel = [(lanes3 + c0) >> 4 for c0 in range(3)]
    sel0 = [v == 0 for v in vsel]
    sel1 = [v == 1 for v in vsel]

    def chunk_body(g, carry):
        row0 = base + g * P
        pltpu.sync_copy(pts_hbm.at[pl.ds(row0 * 3, P * 3)], pbuf)

        for t in range(P // L):
            s = t * L
            sl = pl.ds(s, L)
            v0 = pbuf[pl.ds(t * 48, L)]
            v1 = pbuf[pl.ds(t * 48 + L, L)]
            v2 = pbuf[pl.ds(t * 48 + 2 * L, L)]

            def deint(c0):
                g0 = v0[idxm[c0]]
                g1 = v1[idxm[c0]]
                g2 = v2[idxm[c0]]
                return jnp.where(sel0[c0], g0, jnp.where(sel1[c0], g1, g2))

            zi0, zi1, fz = _axis_coords(deint(0))
            yi0, yi1, fy = _axis_coords(deint(1))
            xi0, xi1, fx = _axis_coords(deint(2))
            zy00 = zi0 * (H * W) + yi0 * W
            zy01 = zi0 * (H * W) + yi1 * W
            zy10 = zi1 * (H * W) + yi0 * W
            zy11 = zi1 * (H * W) + yi1 * W
            idx_v[0, sl] = zy00 + xi0
            idx_v[1, sl] = zy00 + xi1
            idx_v[2, sl] = zy01 + xi0
            idx_v[3, sl] = zy01 + xi1
            idx_v[4, sl] = zy10 + xi0
            idx_v[5, sl] = zy10 + xi1
            idx_v[6, sl] = zy11 + xi0
            idx_v[7, sl] = zy11 + xi1
            fz0 = 1.0 - fz
            fy0 = 1.0 - fy
            fx0 = 1.0 - fx
            m00 = fz0 * fy0
            m01 = fz0 * fy
            m10 = fz * fy0
            m11 = fz * fy
            w8_v[pl.ds(0 * P + s, L)] = m00 * fx0
            w8_v[pl.ds(1 * P + s, L)] = m00 * fx
            w8_v[pl.ds(2 * P + s, L)] = m01 * fx0
            w8_v[pl.ds(3 * P + s, L)] = m01 * fx
            w8_v[pl.ds(4 * P + s, L)] = m10 * fx0
            w8_v[pl.ds(5 * P + s, L)] = m10 * fx
            w8_v[pl.ds(6 * P + s, L)] = m11 * fx0
            w8_v[pl.ds(7 * P + s, L)] = m11 * fx

        copies = [
            pltpu.async_copy(table_hbm.at[idx_v.at[k]],
                             rows_v.at[pl.ds(k * P, P)], gsem)
            for k in range(8)
        ]
        for cp in copies:
            cp.wait()

        def grp_body(t, carry2):
            jbase = t * L
            wrows = [w8_v[pl.ds(k * P + jbase, L)] for k in range(8)]
            for jj in range(L):
                j = jbase + jj
                lane = jnp.full((L,), jj, jnp.int32)
                acc0 = jnp.zeros((L,), jnp.float32)
                acc1 = jnp.zeros((L,), jnp.float32)
                for k in range(8):
                    wb = wrows[k][lane]
                    acc0 = acc0 + wb * rows_v[k * P + j, pl.ds(0, L)]
                    acc1 = acc1 + wb * rows_v[k * P + j, pl.ds(L, L)]
                out_v[j, pl.ds(0, L)] = acc0
                out_v[j, pl.ds(L, L)] = acc1
            return carry2

        lax.fori_loop(0, P // L, grp_body, 0)
        pltpu.sync_copy(out_v, out_hbm.at[pl.ds(row0, P)])
        return carry

    lax.fori_loop(0, NCHUNK, chunk_body, 0)


def kernel(warped_sample_points, voxel_grid):
    t1 = voxel_grid.reshape(C, D * H, W).transpose(1, 0, 2)
    t1 = lax.optimization_barrier(t1)
    table = t1.transpose(0, 2, 1).reshape(DHW, C)
    pts = warped_sample_points.reshape(3 * B)
    return _sc_interp(pts, table)


# speedup vs baseline: 1.5734x; 1.1343x over previous
import functools

import jax
import jax.numpy as jnp
from jax import lax
from jax.experimental import pallas as pl
from jax.experimental.pallas import tpu as pltpu
from jax.experimental.pallas import tpu_sc as plsc

B = 262144
C = 32
D = H = W = 128
DHW = D * H * W

NC = 2
NS = 16
NW = NC * NS
PW = B // NW
P = 128
NCHUNK = PW // P
L = 16


def _axis_coords(p):
    f = ((p + 1.0) * 0.5) * 127.0
    i0 = f.astype(jnp.int32)
    fr = f - i0.astype(jnp.float32)
    i1 = jnp.minimum(i0 + 1, 127)
    return i0, i1, fr


mesh = plsc.VectorSubcoreMesh(core_axis_name="c", subcore_axis_name="s")

@functools.partial(
    pl.kernel,
    out_type=jax.ShapeDtypeStruct((B, C), jnp.float32),
    mesh=mesh,
    scratch_types=[
        pltpu.VMEM((P,), jnp.float32),
        pltpu.VMEM((P,), jnp.float32),
        pltpu.VMEM((P,), jnp.float32),
        pltpu.VMEM((8, P), jnp.int32),
        pltpu.VMEM((8, P), jnp.int32),
        pltpu.VMEM((8 * P,), jnp.float32),
        pltpu.VMEM((8 * P,), jnp.float32),
        pltpu.VMEM((8 * P, C), jnp.float32),
        pltpu.VMEM((8 * P, C), jnp.float32),
        pltpu.VMEM((P, C), jnp.float32),
        pltpu.SemaphoreType.DMA,
        pltpu.SemaphoreType.DMA,
    ],
    compiler_params=pltpu.CompilerParams(use_tc_tiling_on_sc=False),
)
def _sc_interp(pts_hbm, table_hbm, out_hbm,
               zv, yv, xv, idx_v0, idx_v1, w8_v0, w8_v1,
               rows_v0, rows_v1, out_v, gsem0, gsem1):
    wid = lax.axis_index("s") * NC + lax.axis_index("c")
    base = wid * PW

    def phase1(g, idx_v, w8_v):
        row0 = base + g * P
        pltpu.sync_copy(pts_hbm.at[pl.ds(row0, P)], zv)
        pltpu.sync_copy(pts_hbm.at[pl.ds(B + row0, P)], yv)
        pltpu.sync_copy(pts_hbm.at[pl.ds(2 * B + row0, P)], xv)

        for t in range(P // L):
            s = t * L
            sl = pl.ds(s, L)
            zi0, zi1, fz = _axis_coords(zv[sl])
            yi0, yi1, fy = _axis_coords(yv[sl])
            xi0, xi1, fx = _axis_coords(xv[sl])
            zy00 = zi0 * (H * W) + yi0 * W
            zy01 = zi0 * (H * W) + yi1 * W
            zy10 = zi1 * (H * W) + yi0 * W
            zy11 = zi1 * (H * W) + yi1 * W
            idx_v[0, sl] = zy00 + xi0
            idx_v[1, sl] = zy00 + xi1
            idx_v[2, sl] = zy01 + xi0
            idx_v[3, sl] = zy01 + xi1
            idx_v[4, sl] = zy10 + xi0
            idx_v[5, sl] = zy10 + xi1
            idx_v[6, sl] = zy11 + xi0
            idx_v[7, sl] = zy11 + xi1
            fz0 = 1.0 - fz
            fy0 = 1.0 - fy
            fx0 = 1.0 - fx
            m00 = fz0 * fy0
            m01 = fz0 * fy
            m10 = fz * fy0
            m11 = fz * fy
            w8_v[pl.ds(0 * P + s, L)] = m00 * fx0
            w8_v[pl.ds(1 * P + s, L)] = m00 * fx
            w8_v[pl.ds(2 * P + s, L)] = m01 * fx0
            w8_v[pl.ds(3 * P + s, L)] = m01 * fx
            w8_v[pl.ds(4 * P + s, L)] = m10 * fx0
            w8_v[pl.ds(5 * P + s, L)] = m10 * fx
            w8_v[pl.ds(6 * P + s, L)] = m11 * fx0
            w8_v[pl.ds(7 * P + s, L)] = m11 * fx

    def fire(idx_v, rows_v, gsem):
        for k in range(8):
            pltpu.async_copy(table_hbm.at[idx_v.at[k]],
                             rows_v.at[pl.ds(k * P, P)], gsem)

    def drain(idx_v, rows_v, gsem):
        for k in range(8):
            pltpu.make_async_copy(table_hbm.at[idx_v.at[k]],
                                  rows_v.at[pl.ds(k * P, P)], gsem).wait()

    def phase2(g, w8_v, rows_v):
        def grp_body(t, carry2):
            jbase = t * L
            wrows = [w8_v[pl.ds(k * P + jbase, L)] for k in range(8)]
            for jj in range(L):
                j = jbase + jj
                lane = jnp.full((L,), jj, jnp.int32)
                acc0 = jnp.zeros((L,), jnp.float32)
                acc1 = jnp.zeros((L,), jnp.float32)
                for k in range(8):
                    wb = wrows[k][lane]
                    acc0 = acc0 + wb * rows_v[k * P + j, pl.ds(0, L)]
                    acc1 = acc1 + wb * rows_v[k * P + j, pl.ds(L, L)]
                out_v[j, pl.ds(0, L)] = acc0
                out_v[j, pl.ds(L, L)] = acc1
            return carry2

        lax.fori_loop(0, P // L, grp_body, 0)
        pltpu.sync_copy(out_v, out_hbm.at[pl.ds(base + g * P, P)])

    idxs = (idx_v0, idx_v1)
    w8s = (w8_v0, w8_v1)
    rows = (rows_v0, rows_v1)
    sems = (gsem0, gsem1)

    phase1(0, idx_v0, w8_v0)
    fire(idx_v0, rows_v0, gsem0)

    def chunk2(i, carry):
        for b in range(2):
            gg = i * 2 + b

            @pl.when(gg + 1 < NCHUNK)
            def _():
                phase1(gg + 1, idxs[1 - b], w8s[1 - b])
                fire(idxs[1 - b], rows[1 - b], sems[1 - b])

            drain(idxs[b], rows[b], sems[b])
            phase2(gg, w8s[b], rows[b])
        return carry

    lax.fori_loop(0, NCHUNK // 2, chunk2, 0)


def kernel(warped_sample_points, voxel_grid):
    table = voxel_grid[0].transpose(1, 2, 3, 0).reshape(DHW, C)
    pts = warped_sample_points.T.reshape(3 * B)
    return _sc_interp(pts, table)


# speedup vs baseline: 1.5740x; 1.0004x over previous
import functools

import jax
import jax.numpy as jnp
from jax import lax
from jax.experimental import pallas as pl
from jax.experimental.pallas import tpu as pltpu
from jax.experimental.pallas import tpu_sc as plsc

B = 262144
C = 32
D = H = W = 128
DHW = D * H * W

NC = 2
NS = 16
NW = NC * NS
PW = B // NW
P = 128
NCHUNK = PW // P
L = 16


def _axis_coords(p):
    f = ((p + 1.0) * 0.5) * 127.0
    i0 = f.astype(jnp.int32)
    fr = f - i0.astype(jnp.float32)
    i1 = jnp.minimum(i0 + 1, 127)
    return i0, i1, fr


mesh = plsc.VectorSubcoreMesh(core_axis_name="c", subcore_axis_name="s")

@functools.partial(
    pl.kernel,
    out_type=jax.ShapeDtypeStruct((B * C // 128, 128), jnp.float32),
    mesh=mesh,
    scratch_types=[
        pltpu.VMEM((P,), jnp.float32),
        pltpu.VMEM((P,), jnp.float32),
        pltpu.VMEM((P,), jnp.float32),
        pltpu.VMEM((8, P), jnp.int32),
        pltpu.VMEM((8, P), jnp.int32),
        pltpu.VMEM((8 * P,), jnp.float32),
        pltpu.VMEM((8 * P,), jnp.float32),
        pltpu.VMEM((8 * P, C), jnp.float32),
        pltpu.VMEM((8 * P, C), jnp.float32),
        pltpu.VMEM((P * C // 128, 128), jnp.float32),
        pltpu.SemaphoreType.DMA,
        pltpu.SemaphoreType.DMA,
    ],
    compiler_params=pltpu.CompilerParams(use_tc_tiling_on_sc=False),
)
def _sc_interp(pts_hbm, table_hbm, out_hbm,
               zv, yv, xv, idx_v0, idx_v1, w8_v0, w8_v1,
               rows_v0, rows_v1, out_v, gsem0, gsem1):
    wid = lax.axis_index("s") * NC + lax.axis_index("c")
    base = wid * PW

    def phase1(g, idx_v, w8_v):
        row0 = base + g * P
        pltpu.sync_copy(pts_hbm.at[pl.ds(row0, P)], zv)
        pltpu.sync_copy(pts_hbm.at[pl.ds(B + row0, P)], yv)
        pltpu.sync_copy(pts_hbm.at[pl.ds(2 * B + row0, P)], xv)

        for t in range(P // L):
            s = t * L
            sl = pl.ds(s, L)
            zi0, zi1, fz = _axis_coords(zv[sl])
            yi0, yi1, fy = _axis_coords(yv[sl])
            xi0, xi1, fx = _axis_coords(xv[sl])
            zy00 = zi0 * (H * W) + yi0 * W
            zy01 = zi0 * (H * W) + yi1 * W
            zy10 = zi1 * (H * W) + yi0 * W
            zy11 = zi1 * (H * W) + yi1 * W
            idx_v[0, sl] = zy00 + xi0
            idx_v[1, sl] = zy00 + xi1
            idx_v[2, sl] = zy01 + xi0
            idx_v[3, sl] = zy01 + xi1
            idx_v[4, sl] = zy10 + xi0
            idx_v[5, sl] = zy10 + xi1
            idx_v[6, sl] = zy11 + xi0
            idx_v[7, sl] = zy11 + xi1
            fz0 = 1.0 - fz
            fy0 = 1.0 - fy
            fx0 = 1.0 - fx
            m00 = fz0 * fy0
            m01 = fz0 * fy
            m10 = fz * fy0
            m11 = fz * fy
            w8_v[pl.ds(0 * P + s, L)] = m00 * fx0
            w8_v[pl.ds(1 * P + s, L)] = m00 * fx
            w8_v[pl.ds(2 * P + s, L)] = m01 * fx0
            w8_v[pl.ds(3 * P + s, L)] = m01 * fx
            w8_v[pl.ds(4 * P + s, L)] = m10 * fx0
            w8_v[pl.ds(5 * P + s, L)] = m10 * fx
            w8_v[pl.ds(6 * P + s, L)] = m11 * fx0
            w8_v[pl.ds(7 * P + s, L)] = m11 * fx

    def fire(idx_v, rows_v, gsem):
        for k in range(8):
            pltpu.async_copy(table_hbm.at[idx_v.at[k]],
                             rows_v.at[pl.ds(k * P, P)], gsem)

    def drain(idx_v, rows_v, gsem):
        for k in range(8):
            pltpu.make_async_copy(table_hbm.at[idx_v.at[k]],
                                  rows_v.at[pl.ds(k * P, P)], gsem).wait()

    def phase2(g, w8_v, rows_v):
        def grp_body(t, carry2):
            jbase = t * L
            wrows = [w8_v[pl.ds(k * P + jbase, L)] for k in range(8)]
            for jj in range(L):
                j = jbase + jj
                lane = jnp.full((L,), jj, jnp.int32)
                acc0 = jnp.zeros((L,), jnp.float32)
                acc1 = jnp.zeros((L,), jnp.float32)
                for k in range(8):
                    wb = wrows[k][lane]
                    acc0 = acc0 + wb * rows_v[k * P + j, pl.ds(0, L)]
                    acc1 = acc1 + wb * rows_v[k * P + j, pl.ds(L, L)]
                row = t * 4 + (jj >> 2)
                col = (jj & 3) * C
                out_v[row, pl.ds(col, L)] = acc0
                out_v[row, pl.ds(col + L, L)] = acc1
            return carry2

        lax.fori_loop(0, P // L, grp_body, 0)
        pltpu.sync_copy(out_v, out_hbm.at[pl.ds((base + g * P) * C // 128, P * C // 128)])

    idxs = (idx_v0, idx_v1)
    w8s = (w8_v0, w8_v1)
    rows = (rows_v0, rows_v1)
    sems = (gsem0, gsem1)

    phase1(0, idx_v0, w8_v0)
    fire(idx_v0, rows_v0, gsem0)

    def chunk2(i, carry):
        for b in range(2):
            gg = i * 2 + b

            @pl.when(gg + 1 < NCHUNK)
            def _():
                phase1(gg + 1, idxs[1 - b], w8s[1 - b])
                fire(idxs[1 - b], rows[1 - b], sems[1 - b])

            drain(idxs[b], rows[b], sems[b])
            phase2(gg, w8s[b], rows[b])
        return carry

    lax.fori_loop(0, NCHUNK // 2, chunk2, 0)


def kernel(warped_sample_points, voxel_grid):
    table = voxel_grid[0].transpose(1, 2, 3, 0).reshape(DHW, C)
    pts = warped_sample_points.T.reshape(3 * B)
    return _sc_interp(pts, table).reshape(B, C)
